# baseline (speedup 1.0000x reference)
import functools

import jax
import jax.numpy as jnp
from jax import lax
from jax.experimental import pallas as pl
from jax.experimental.pallas import tpu as pltpu
from jax.experimental.pallas import tpu_sc as plsc

NM = 1_000_000
EMBED = 32
BATCH = 16384
NC, NS, L = 2, 16, 16
NW = NC * NS
PAIRS_W = BATCH // NW
IDX_W = 2 * PAIRS_W
CHUNK = 128
NCHUNK = IDX_W // CHUNK
GROUPS = PAIRS_W // L

_mesh = plsc.VectorSubcoreMesh(core_axis_name="c", subcore_axis_name="s")


@functools.partial(
    pl.kernel,
    out_type=jax.ShapeDtypeStruct((BATCH,), jnp.float32),
    mesh=_mesh,
    scratch_types=[
        pltpu.VMEM((IDX_W,), jnp.int32),
        pltpu.VMEM((IDX_W, EMBED), jnp.float32),
        pltpu.VMEM((IDX_W,), jnp.float32),
        pltpu.VMEM((L * L,), jnp.float32),
        pltpu.VMEM((PAIRS_W,), jnp.float32),
        pltpu.SemaphoreType.DMA,
        pltpu.SemaphoreType.DMA,
    ],
    compiler_params=pltpu.CompilerParams(
        needs_layout_passes=False, use_tc_tiling_on_sc=False),
)
def _cf_kernel(x_hbm, bias_hbm, ent_hbm, out_hbm,
               idx_v, rows_v, bias_v, pmat, out_v, sem_e, sem_b):
    wid = lax.axis_index("s") * NC + lax.axis_index("c")
    pltpu.sync_copy(x_hbm.at[pl.ds(wid * IDX_W, IDX_W)], idx_v)

    copies = []
    for c in range(NCHUNK):
        sl = pl.ds(c * CHUNK, CHUNK)
        copies.append(
            pltpu.async_copy(ent_hbm.at[idx_v.at[sl]], rows_v.at[sl], sem_e))
        copies.append(
            pltpu.async_copy(bias_hbm.at[idx_v.at[sl]], bias_v.at[sl], sem_b))
    for cp in copies:
        cp.wait()

    lanes = lax.iota(jnp.int32, L)

    def group_body(g, carry):
        for i in range(L):
            r = 2 * (g * L + i)
            u0 = rows_v[r, pl.ds(0, L)]
            u1 = rows_v[r, pl.ds(L, L)]
            v0 = rows_v[r + 1, pl.ds(0, L)]
            v1 = rows_v[r + 1, pl.ds(L, L)]
            pmat[pl.ds(i * L, L)] = u0 * v0 + u1 * v1
        acc = jnp.zeros((L,), jnp.float32)
        row = lanes * L
        for j in range(L):
            acc = acc + plsc.load_gather(pmat, [row + j])
        pair = g * L + lanes
        acc = acc + plsc.load_gather(bias_v, [2 * pair])
        acc = acc + plsc.load_gather(bias_v, [2 * pair + 1])
        out_v[pl.ds(g * L, L)] = acc
        return carry

    lax.fori_loop(0, GROUPS, group_body, 0)
    pltpu.sync_copy(out_v, out_hbm.at[pl.ds(wid * PAIRS_W, PAIRS_W)])


def kernel(x, biases, entities):
    xf = x.reshape(-1).astype(jnp.int32)
    bf = biases.reshape(-1)
    return _cf_kernel(xf, bf, entities)


# speedup vs baseline: 1.0034x; 1.0034x over previous
import functools

import jax
import jax.numpy as jnp
from jax import lax
from jax.experimental import pallas as pl
from jax.experimental.pallas import tpu as pltpu
from jax.experimental.pallas import tpu_sc as plsc

NM = 1_000_000
EMBED = 32
BATCH = 16384
NC, NS, L = 2, 16, 16
NW = NC * NS
PAIRS_W = BATCH // NW
SLOTS_W = 2 * PAIRS_W
CHUNK = 128
GROUPS = PAIRS_W // L

_mesh = plsc.VectorSubcoreMesh(core_axis_name="c", subcore_axis_name="s")


@functools.partial(
    pl.kernel,
    out_type=jax.ShapeDtypeStruct((BATCH,), jnp.float32),
    mesh=_mesh,
    scratch_types=[
        pltpu.VMEM((SLOTS_W,), jnp.int32),
        pltpu.VMEM((SLOTS_W, EMBED), jnp.float32),
        pltpu.VMEM((SLOTS_W,), jnp.float32),
        pltpu.VMEM((L * L,), jnp.float32),
        pltpu.VMEM((PAIRS_W,), jnp.float32),
        pltpu.SemaphoreType.DMA,
        pltpu.SemaphoreType.DMA,
    ],
    compiler_params=pltpu.CompilerParams(
        needs_layout_passes=False, use_tc_tiling_on_sc=False),
)
def _cf_kernel(u_hbm, v_hbm, bias_hbm, ent_hbm, out_hbm,
               uv_idx, rows, bvals, pmat, out_v, sem_e, sem_b):
    wid = lax.axis_index("s") * NC + lax.axis_index("c")
    base = wid * PAIRS_W
    pltpu.sync_copy(u_hbm.at[pl.ds(base, PAIRS_W)],
                    uv_idx.at[pl.ds(0, PAIRS_W)])
    pltpu.sync_copy(v_hbm.at[pl.ds(base, PAIRS_W)],
                    uv_idx.at[pl.ds(PAIRS_W, PAIRS_W)])

    copies = []
    for c in range(SLOTS_W // CHUNK):
        sl = pl.ds(c * CHUNK, CHUNK)
        copies.append(
            pltpu.async_copy(ent_hbm.at[uv_idx.at[sl]], rows.at[sl], sem_e))
        copies.append(
            pltpu.async_copy(bias_hbm.at[uv_idx.at[sl]], bvals.at[sl], sem_b))
    for cp in copies:
        cp.wait()

    lanes = lax.iota(jnp.int32, L)

    def group_body(g, carry):
        for i in range(L):
            ju = g * L + i
            jv = PAIRS_W + g * L + i
            u0 = rows[ju, pl.ds(0, L)]
            u1 = rows[ju, pl.ds(L, L)]
            v0 = rows[jv, pl.ds(0, L)]
            v1 = rows[jv, pl.ds(L, L)]
            pmat[pl.ds(i * L, L)] = u0 * v0 + u1 * v1
        acc = jnp.zeros((L,), jnp.float32)
        row = lanes * L
        for j in range(L):
            acc = acc + plsc.load_gather(pmat, [row + j])
        acc = acc + bvals[pl.ds(g * L, L)]
        acc = acc + bvals[pl.ds(PAIRS_W + g * L, L)]
        out_v[pl.ds(g * L, L)] = acc
        return carry

    lax.fori_loop(0, GROUPS, group_body, 0)
    pltpu.sync_copy(out_v, out_hbm.at[pl.ds(base, PAIRS_W)])


def kernel(x, biases, entities):
    x = x.astype(jnp.int32)
    return _cf_kernel(x[:, 0], x[:, 1], biases.reshape(-1), entities)
